# Initial kernel scaffold; baseline (speedup 1.0000x reference)
import jax
import jax.numpy as jnp
from jax.experimental import pallas as pl


def kernel(x, edge_index, edge_weight, batch, W1, b1, gamma1, beta1, W2, b2, gamma2, beta2, linW, linb):
    raise NotImplementedError("write your pallas kernel here")


# speedup vs baseline: 2.7946x; 2.7946x over previous
import functools
import jax
import jax.numpy as jnp
from jax.experimental import pallas as pl
from jax.experimental.pallas import tpu as pltpu

N = 10000
E = 320000
G = 64
F = 128
OUT_F = 10
EPS = 1e-5
BLK = 1000
NBLK = N // BLK
NEG = -3.0e38


def _mm_body(x_ref, w_ref, o_ref):
    o_ref[...] = jnp.dot(x_ref[...], w_ref[...],
                         preferred_element_type=jnp.float32)


def _mm(x, W):
    return pl.pallas_call(
        _mm_body,
        grid=(NBLK,),
        in_specs=[pl.BlockSpec((BLK, F), lambda i: (i, 0)),
                  pl.BlockSpec((F, F), lambda i: (0, 0))],
        out_specs=pl.BlockSpec((BLK, F), lambda i: (i, 0)),
        out_shape=jax.ShapeDtypeStruct((N, F), jnp.float32),
    )(x, W)


def _scale_body(h_ref, d_ref, o_ref):
    o_ref[...] = h_ref[...] * d_ref[...]


def _scale(h, dis):
    return pl.pallas_call(
        _scale_body,
        grid=(NBLK,),
        in_specs=[pl.BlockSpec((BLK, F), lambda i: (i, 0)),
                  pl.BlockSpec((BLK, 1), lambda i: (i, 0))],
        out_specs=pl.BlockSpec((BLK, F), lambda i: (i, 0)),
        out_shape=jax.ShapeDtypeStruct((N, F), jnp.float32),
    )(h, dis)


def _combine_mm_body(s_ref, hp_ref, d_ref, b_ref, a_ref, be_ref, w_ref, o_ref):
    d = d_ref[...]
    z = d * (s_ref[...] + hp_ref[...]) + b_ref[...]
    y = a_ref[...] * jax.nn.relu(z) + be_ref[...]
    o_ref[...] = d * jnp.dot(y, w_ref[...], preferred_element_type=jnp.float32)


def _combine_mm(S, hp, dis, b, a, beta, W):
    return pl.pallas_call(
        _combine_mm_body,
        grid=(NBLK,),
        in_specs=[pl.BlockSpec((BLK, F), lambda i: (i, 0)),
                  pl.BlockSpec((BLK, F), lambda i: (i, 0)),
                  pl.BlockSpec((BLK, 1), lambda i: (i, 0)),
                  pl.BlockSpec((1, F), lambda i: (0, 0)),
                  pl.BlockSpec((1, F), lambda i: (0, 0)),
                  pl.BlockSpec((1, F), lambda i: (0, 0)),
                  pl.BlockSpec((F, F), lambda i: (0, 0))],
        out_specs=pl.BlockSpec((BLK, F), lambda i: (i, 0)),
        out_shape=jax.ShapeDtypeStruct((N, F), jnp.float32),
    )(S, hp, dis, b.reshape(1, F), a.reshape(1, F), beta.reshape(1, F), W)


def _pool_body(s_ref, hp_ref, d_ref, b_ref, a_ref, be_ref, batch_ref,
               lw_ref, lb_ref, o_ref, sum_ref, cnt_ref, mx_ref):
    i = pl.program_id(0)

    @pl.when(i == 0)
    def _():
        sum_ref[...] = jnp.zeros_like(sum_ref)
        cnt_ref[...] = jnp.zeros_like(cnt_ref)
        mx_ref[...] = jnp.full_like(mx_ref, NEG)

    d = d_ref[...]
    z = d * (s_ref[...] + hp_ref[...]) + b_ref[...]
    y = a_ref[...] * jax.nn.relu(z) + be_ref[...]

    batch_row = batch_ref[0]
    seg = jax.lax.broadcasted_iota(jnp.int32, (G, BLK), 0)
    m = (batch_row == seg).astype(jnp.float32)
    sum_ref[...] += jnp.dot(m, y, preferred_element_type=jnp.float32)
    cnt_ref[...] += jnp.sum(m, axis=1, keepdims=True)

    batch_col = batch_row.reshape(BLK, 1)
    lo = batch_ref[0, 0, 0]
    hi = batch_ref[0, 0, BLK - 1]

    def seg_max(g, _):
        sel = jnp.where(batch_col == g, y, NEG)
        blkmax = jnp.max(sel, axis=0, keepdims=True)
        cur = mx_ref[pl.ds(g, 1), :]
        mx_ref[pl.ds(g, 1), :] = jnp.maximum(cur, blkmax)
        return 0

    jax.lax.fori_loop(lo, hi + 1, seg_max, 0)

    @pl.when(i == NBLK - 1)
    def _():
        cnt = cnt_ref[...]
        s = sum_ref[...]
        mean = s / jnp.maximum(cnt, 1.0)
        mx = jnp.where(cnt > 0, mx_ref[...], 0.0)
        pooled = jnp.concatenate([s, mean, mx], axis=-1)
        o_ref[...] = (jnp.dot(pooled, lw_ref[...],
                              preferred_element_type=jnp.float32)
                      + lb_ref[...])


def _pool_readout(S, hp, dis, b, a, beta, batch, linW, linb):
    batch3 = batch.reshape(NBLK, 1, BLK)
    return pl.pallas_call(
        _pool_body,
        grid=(NBLK,),
        in_specs=[pl.BlockSpec((BLK, F), lambda i: (i, 0)),
                  pl.BlockSpec((BLK, F), lambda i: (i, 0)),
                  pl.BlockSpec((BLK, 1), lambda i: (i, 0)),
                  pl.BlockSpec((1, F), lambda i: (0, 0)),
                  pl.BlockSpec((1, F), lambda i: (0, 0)),
                  pl.BlockSpec((1, F), lambda i: (0, 0)),
                  pl.BlockSpec((1, 1, BLK), lambda i: (i, 0, 0)),
                  pl.BlockSpec((3 * F, OUT_F), lambda i: (0, 0)),
                  pl.BlockSpec((1, OUT_F), lambda i: (0, 0))],
        out_specs=pl.BlockSpec((G, OUT_F), lambda i: (0, 0)),
        out_shape=jax.ShapeDtypeStruct((G, OUT_F), jnp.float32),
        scratch_shapes=[pltpu.VMEM((G, F), jnp.float32),
                        pltpu.VMEM((G, 1), jnp.float32),
                        pltpu.VMEM((G, F), jnp.float32)],
    )(S, hp, dis, b.reshape(1, F), a.reshape(1, F), beta.reshape(1, F),
      batch3, linW, linb.reshape(1, OUT_F))


def _deg_scatter(col, ew):
    return jnp.zeros((N,), jnp.float32).at[col].add(ew) + 1.0


def _spmm_scatter(hp, row, col, ew):
    return jnp.zeros((N, F), jnp.float32).at[col].add(ew[:, None] * hp[row])


def kernel(x, edge_index, edge_weight, batch, W1, b1, gamma1, beta1,
           W2, b2, gamma2, beta2, linW, linb):
    row = edge_index[0]
    col = edge_index[1]
    bnscale = jax.lax.rsqrt(jnp.float32(1.0 + EPS))
    a1 = gamma1 * bnscale
    a2 = gamma2 * bnscale

    deg = _deg_scatter(col, edge_weight)
    dis = jax.lax.rsqrt(deg).reshape(N, 1)

    hw1 = _mm(x, W1)
    h1p = _scale(hw1, dis)
    S1 = _spmm_scatter(h1p, row, col, edge_weight)
    h2p = _combine_mm(S1, h1p, dis, b1, a1, beta1, W2)
    S2 = _spmm_scatter(h2p, row, col, edge_weight)
    return _pool_readout(S2, h2p, dis, b2, a2, beta2, batch, linW, linb)


# speedup vs baseline: 4.1325x; 1.4787x over previous
import dataclasses
import functools
import jax
import jax.numpy as jnp
from jax.experimental import pallas as pl
from jax.experimental.pallas import tpu as pltpu
from jax.experimental.pallas import tpu_sc as plsc

N = 10000
E = 320000
G = 64
F = 128
FH = 64
OUT_F = 10
EPS = 1e-5
BLK = 1000
NBLK = N // BLK
NEG = -3.0e38


def _mm_body(x_ref, w_ref, o_ref):
    o_ref[...] = jnp.dot(x_ref[...], w_ref[...],
                         preferred_element_type=jnp.float32)


def _mm(x, W):
    return pl.pallas_call(
        _mm_body,
        grid=(NBLK,),
        in_specs=[pl.BlockSpec((BLK, F), lambda i: (i, 0)),
                  pl.BlockSpec((F, F), lambda i: (0, 0))],
        out_specs=pl.BlockSpec((BLK, F), lambda i: (i, 0)),
        out_shape=jax.ShapeDtypeStruct((N, F), jnp.float32),
    )(x, W)


def _scale_body(h_ref, d_ref, o_ref):
    o_ref[...] = h_ref[...] * d_ref[...]


def _scale(h, dis):
    return pl.pallas_call(
        _scale_body,
        grid=(NBLK,),
        in_specs=[pl.BlockSpec((BLK, F), lambda i: (i, 0)),
                  pl.BlockSpec((BLK, 1), lambda i: (i, 0))],
        out_specs=pl.BlockSpec((BLK, F), lambda i: (i, 0)),
        out_shape=jax.ShapeDtypeStruct((N, F), jnp.float32),
    )(h, dis)


def _combine_mm_body(s_ref, hp_ref, d_ref, b_ref, a_ref,
                     be_ref, w_ref, o_ref):
    d = d_ref[...]
    s = jnp.concatenate([s_ref[0], s_ref[1]], axis=-1)
    z = d * (s + hp_ref[...]) + b_ref[...]
    y = a_ref[...] * jax.nn.relu(z) + be_ref[...]
    o_ref[...] = d * jnp.dot(y, w_ref[...], preferred_element_type=jnp.float32)


def _combine_mm(S, hp, dis, b, a, beta, W):
    return pl.pallas_call(
        _combine_mm_body,
        grid=(NBLK,),
        in_specs=[pl.BlockSpec((2, BLK, FH), lambda i: (0, i, 0)),
                  pl.BlockSpec((BLK, F), lambda i: (i, 0)),
                  pl.BlockSpec((BLK, 1), lambda i: (i, 0)),
                  pl.BlockSpec((1, F), lambda i: (0, 0)),
                  pl.BlockSpec((1, F), lambda i: (0, 0)),
                  pl.BlockSpec((1, F), lambda i: (0, 0)),
                  pl.BlockSpec((F, F), lambda i: (0, 0))],
        out_specs=pl.BlockSpec((BLK, F), lambda i: (i, 0)),
        out_shape=jax.ShapeDtypeStruct((N, F), jnp.float32),
    )(S, hp, dis, b.reshape(1, F), a.reshape(1, F),
      beta.reshape(1, F), W)


def _pool_body(s_ref, hp_ref, d_ref, b_ref, a_ref, be_ref,
               batch_ref, lw_ref, lb_ref, o_ref, sum_ref, cnt_ref, mx_ref):
    i = pl.program_id(0)

    @pl.when(i == 0)
    def _():
        sum_ref[...] = jnp.zeros_like(sum_ref)
        cnt_ref[...] = jnp.zeros_like(cnt_ref)
        mx_ref[...] = jnp.full_like(mx_ref, NEG)

    d = d_ref[...]
    s = jnp.concatenate([s_ref[0], s_ref[1]], axis=-1)
    z = d * (s + hp_ref[...]) + b_ref[...]
    y = a_ref[...] * jax.nn.relu(z) + be_ref[...]

    batch_row = batch_ref[0]
    seg = jax.lax.broadcasted_iota(jnp.int32, (G, BLK), 0)
    m = (batch_row == seg).astype(jnp.float32)
    sum_ref[...] += jnp.dot(m, y, preferred_element_type=jnp.float32)
    cnt_ref[...] += jnp.sum(m, axis=1, keepdims=True)

    batch_col = batch_row.reshape(BLK, 1)
    lo = batch_ref[0, 0, 0]
    hi = batch_ref[0, 0, BLK - 1]

    def seg_max(g, _):
        sel = jnp.where(batch_col == g, y, NEG)
        blkmax = jnp.max(sel, axis=0, keepdims=True)
        cur = mx_ref[pl.ds(g, 1), :]
        mx_ref[pl.ds(g, 1), :] = jnp.maximum(cur, blkmax)
        return 0

    jax.lax.fori_loop(lo, hi + 1, seg_max, 0)

    @pl.when(i == NBLK - 1)
    def _():
        cnt = cnt_ref[...]
        ssum = sum_ref[...]
        mean = ssum / jnp.maximum(cnt, 1.0)
        mx = jnp.where(cnt > 0, mx_ref[...], 0.0)
        pooled = jnp.concatenate([ssum, mean, mx], axis=-1)
        o_ref[...] = (jnp.dot(pooled, lw_ref[...],
                              preferred_element_type=jnp.float32)
                      + lb_ref[...])


def _pool_readout(S, hp, dis, b, a, beta, batch, linW, linb):
    batch3 = batch.reshape(NBLK, 1, BLK)
    return pl.pallas_call(
        _pool_body,
        grid=(NBLK,),
        in_specs=[pl.BlockSpec((2, BLK, FH), lambda i: (0, i, 0)),
                  pl.BlockSpec((BLK, F), lambda i: (i, 0)),
                  pl.BlockSpec((BLK, 1), lambda i: (i, 0)),
                  pl.BlockSpec((1, F), lambda i: (0, 0)),
                  pl.BlockSpec((1, F), lambda i: (0, 0)),
                  pl.BlockSpec((1, F), lambda i: (0, 0)),
                  pl.BlockSpec((1, 1, BLK), lambda i: (i, 0, 0)),
                  pl.BlockSpec((3 * F, OUT_F), lambda i: (0, 0)),
                  pl.BlockSpec((1, OUT_F), lambda i: (0, 0))],
        out_specs=pl.BlockSpec((G, OUT_F), lambda i: (0, 0)),
        out_shape=jax.ShapeDtypeStruct((G, OUT_F), jnp.float32),
        scratch_shapes=[pltpu.VMEM((G, F), jnp.float32),
                        pltpu.VMEM((G, 1), jnp.float32),
                        pltpu.VMEM((G, F), jnp.float32)],
    )(S, hp, dis, b.reshape(1, F), a.reshape(1, F),
      beta.reshape(1, F), batch3, linW, linb.reshape(1, OUT_F))


_SC_MESH = plsc.VectorSubcoreMesh(core_axis_name="c", subcore_axis_name="s",
                                  num_cores=2, num_subcores=16)
_SC_CP = pltpu.CompilerParams()
if "needs_layout_passes" in pltpu.CompilerParams.__dataclass_fields__:
    _SC_CP = dataclasses.replace(_SC_CP, needs_layout_passes=False)

NCORE = 2
NSUB = 16
NTILES = NCORE * NSUB
EPT = E // NTILES
DCH = 80
NPAD = 10240
RPS = NPAD // NSUB
ZR = 128
SCH = 80
EPS_SUB = E // NSUB
NCHUNK = EPS_SUB // SCH


def _deg_body(col_hbm, ew_hbm, out_hbm, cid_v, ew_v, pay_v, zb_v, acc_sh):
    ci = jax.lax.axis_index("c")
    si = jax.lax.axis_index("s")
    wid = si * NCORE + ci

    zv = jnp.zeros((16,), jnp.float32)
    for i in range(ZR):
        zb_v[i, :] = zv
    for j in range(RPS // ZR):
        pltpu.sync_copy(zb_v, acc_sh.at[pl.ds(si * RPS + j * ZR, ZR)])
    plsc.subcore_barrier()

    base = wid * EPT

    @pl.loop(0, EPT // DCH)
    def _(g):
        off = base + g * DCH
        pltpu.sync_copy(col_hbm.at[pl.ds(off, DCH)], cid_v)
        pltpu.sync_copy(ew_hbm.at[pl.ds(off, DCH)], ew_v)
        for i in range(DCH):
            w = plsc.load_gather(ew_v, [jnp.full((16,), i, jnp.int32)])
            pay_v[i, :] = w
        pltpu.sync_copy(pay_v, acc_sh.at[cid_v], add=True)

    plsc.subcore_barrier()
    pltpu.sync_copy(acc_sh.at[pl.ds(si * RPS, RPS)],
                    out_hbm.at[ci].at[pl.ds(si * RPS, RPS)])


def _deg_scatter(col, ew):
    deg_sc = pl.kernel(
        _deg_body,
        out_type=jax.ShapeDtypeStruct((NCORE, NPAD, 16), jnp.float32),
        mesh=_SC_MESH,
        scratch_types=[
            pltpu.VMEM((DCH,), jnp.int32),
            pltpu.VMEM((DCH,), jnp.float32),
            pltpu.VMEM((DCH, 16), jnp.float32),
            pltpu.VMEM((ZR, 16), jnp.float32),
            pltpu.VMEM_SHARED((NPAD, 16), jnp.float32),
        ],
        compiler_params=_SC_CP,
    )
    partials = deg_sc(col, ew)
    return partials[0, :N, 0] + partials[1, :N, 0] + 1.0


def _spmm_body(hp_hbm, row_hbm, col_hbm, ew_hbm, z_hbm, out_hbm,
               rid_v, cid_v, ew_v, msg_v, msgh_v, acc_sh, gsem):
    ci = jax.lax.axis_index("c")
    si = jax.lax.axis_index("s")
    coff = ci * FH

    pltpu.sync_copy(z_hbm, acc_sh.at[pl.ds(si * RPS, RPS)])
    plsc.subcore_barrier()

    @pl.loop(0, NCHUNK)
    def _(g):
        pltpu.sync_copy(row_hbm.at[si].at[g], rid_v)
        pltpu.sync_copy(col_hbm.at[si].at[g], cid_v)
        pltpu.sync_copy(ew_hbm.at[si].at[g], ew_v)
        pltpu.async_copy(hp_hbm.at[rid_v], msg_v, gsem).wait()
        for i in range(SCH):
            w = plsc.load_gather(ew_v, [jnp.full((16,), i, jnp.int32)])
            for b in range(FH // 16):
                msgh_v[i, pl.ds(b * 16, 16)] = (
                    msg_v[i, pl.ds(coff + b * 16, 16)] * w)
        pltpu.sync_copy(msgh_v, acc_sh.at[cid_v], add=True)

    plsc.subcore_barrier()
    pltpu.sync_copy(acc_sh.at[pl.ds(si * RPS, RPS)],
                    out_hbm.at[ci].at[pl.ds(si * RPS, RPS)])


def _spmm_sc(hp, row3, col3, ew3, zrows):
    spmm = pl.kernel(
        _spmm_body,
        out_type=jax.ShapeDtypeStruct((NCORE, NPAD, FH), jnp.float32),
        mesh=_SC_MESH,
        scratch_types=[
            pltpu.VMEM((SCH,), jnp.int32),
            pltpu.VMEM((SCH,), jnp.int32),
            pltpu.VMEM((SCH,), jnp.float32),
            pltpu.VMEM((SCH, F), jnp.float32),
            pltpu.VMEM((SCH, FH), jnp.float32),
            pltpu.VMEM_SHARED((NPAD, FH), jnp.float32),
            pltpu.SemaphoreType.DMA,
        ],
        compiler_params=_SC_CP,
    )
    return spmm(hp, row3, col3, ew3, zrows)


def kernel(x, edge_index, edge_weight, batch, W1, b1, gamma1, beta1,
           W2, b2, gamma2, beta2, linW, linb):
    row = edge_index[0]
    col = edge_index[1]
    bnscale = jax.lax.rsqrt(jnp.float32(1.0 + EPS))
    a1 = gamma1 * bnscale
    a2 = gamma2 * bnscale

    deg = _deg_scatter(col, edge_weight)
    dis = jax.lax.rsqrt(deg).reshape(N, 1)

    row3 = row.reshape(NSUB, NCHUNK, SCH)
    col3 = col.reshape(NSUB, NCHUNK, SCH)
    ew3 = edge_weight.reshape(NSUB, NCHUNK, SCH)
    zrows = jnp.zeros((RPS, FH), jnp.float32)

    hw1 = _mm(x, W1)
    h1p = _scale(hw1, dis)

    pstack = (jnp.stack([b1, b2]), jnp.stack([a1, a2]),
              jnp.stack([beta1, beta2]))

    def layer_body(hp, params):
        b_, a_, be_ = params
        S = _spmm_sc(hp, row3, col3, ew3, zrows)
        hp_next = _combine_mm(S, hp, dis, b_, a_, be_, W2)
        return hp_next, (S, hp)

    _, (Ss, hps) = jax.lax.scan(layer_body, h1p, pstack)
    S2 = Ss[1]
    h2p = hps[1]
    return _pool_readout(S2, h2p, dis, b2, a2, beta2, batch, linW, linb)


# speedup vs baseline: 5.6703x; 1.3721x over previous
import dataclasses
import functools
import jax
import jax.numpy as jnp
from jax.experimental import pallas as pl
from jax.experimental.pallas import tpu as pltpu
from jax.experimental.pallas import tpu_sc as plsc

N = 10000
E = 320000
G = 64
F = 128
FH = 64
OUT_F = 10
EPS = 1e-5
BLK = 1000
NBLK = N // BLK
NEG = -3.0e38


def _mm_body(x_ref, w_ref, o_ref):
    o_ref[...] = jnp.dot(x_ref[...], w_ref[...],
                         preferred_element_type=jnp.float32)


def _mm(x, W):
    return pl.pallas_call(
        _mm_body,
        grid=(NBLK,),
        in_specs=[pl.BlockSpec((BLK, F), lambda i: (i, 0)),
                  pl.BlockSpec((F, F), lambda i: (0, 0))],
        out_specs=pl.BlockSpec((BLK, F), lambda i: (i, 0)),
        out_shape=jax.ShapeDtypeStruct((N, F), jnp.float32),
    )(x, W)


def _scale_body(h_ref, d_ref, o_ref):
    o_ref[...] = h_ref[...] * d_ref[...]


def _scale(h, dis):
    return pl.pallas_call(
        _scale_body,
        grid=(NBLK,),
        in_specs=[pl.BlockSpec((BLK, F), lambda i: (i, 0)),
                  pl.BlockSpec((BLK, 1), lambda i: (i, 0))],
        out_specs=pl.BlockSpec((BLK, F), lambda i: (i, 0)),
        out_shape=jax.ShapeDtypeStruct((N, F), jnp.float32),
    )(h, dis)


def _combine_mm_body(s_ref, hp_ref, d_ref, b_ref, a_ref,
                     be_ref, w_ref, o_ref):
    d = d_ref[...]
    s = jnp.concatenate([s_ref[0], s_ref[1]], axis=-1)
    z = d * (s + hp_ref[...]) + b_ref[...]
    y = a_ref[...] * jax.nn.relu(z) + be_ref[...]
    o_ref[...] = d * jnp.dot(y, w_ref[...], preferred_element_type=jnp.float32)


def _combine_mm(S, hp, dis, b, a, beta, W):
    return pl.pallas_call(
        _combine_mm_body,
        grid=(NBLK,),
        in_specs=[pl.BlockSpec((2, BLK, FH), lambda i: (0, i, 0)),
                  pl.BlockSpec((BLK, F), lambda i: (i, 0)),
                  pl.BlockSpec((BLK, 1), lambda i: (i, 0)),
                  pl.BlockSpec((1, F), lambda i: (0, 0)),
                  pl.BlockSpec((1, F), lambda i: (0, 0)),
                  pl.BlockSpec((1, F), lambda i: (0, 0)),
                  pl.BlockSpec((F, F), lambda i: (0, 0))],
        out_specs=pl.BlockSpec((BLK, F), lambda i: (i, 0)),
        out_shape=jax.ShapeDtypeStruct((N, F), jnp.float32),
    )(S, hp, dis, b.reshape(1, F), a.reshape(1, F),
      beta.reshape(1, F), W)


def _pool_body(s_ref, hp_ref, d_ref, b_ref, a_ref, be_ref,
               batch_ref, lw_ref, lb_ref, o_ref, sum_ref, cnt_ref, mx_ref):
    i = pl.program_id(0)

    @pl.when(i == 0)
    def _():
        sum_ref[...] = jnp.zeros_like(sum_ref)
        cnt_ref[...] = jnp.zeros_like(cnt_ref)
        mx_ref[...] = jnp.full_like(mx_ref, NEG)

    d = d_ref[...]
    s = jnp.concatenate([s_ref[0], s_ref[1]], axis=-1)
    z = d * (s + hp_ref[...]) + b_ref[...]
    y = a_ref[...] * jax.nn.relu(z) + be_ref[...]

    batch_row = batch_ref[0]
    seg = jax.lax.broadcasted_iota(jnp.int32, (G, BLK), 0)
    m = (batch_row == seg).astype(jnp.float32)
    sum_ref[...] += jnp.dot(m, y, preferred_element_type=jnp.float32)
    cnt_ref[...] += jnp.sum(m, axis=1, keepdims=True)

    batch_col = batch_row.reshape(BLK, 1)
    lo = batch_ref[0, 0, 0]
    hi = batch_ref[0, 0, BLK - 1]

    def seg_max(g, _):
        sel = jnp.where(batch_col == g, y, NEG)
        blkmax = jnp.max(sel, axis=0, keepdims=True)
        cur = mx_ref[pl.ds(g, 1), :]
        mx_ref[pl.ds(g, 1), :] = jnp.maximum(cur, blkmax)
        return 0

    jax.lax.fori_loop(lo, hi + 1, seg_max, 0)

    @pl.when(i == NBLK - 1)
    def _():
        cnt = cnt_ref[...]
        ssum = sum_ref[...]
        mean = ssum / jnp.maximum(cnt, 1.0)
        mx = jnp.where(cnt > 0, mx_ref[...], 0.0)
        pooled = jnp.concatenate([ssum, mean, mx], axis=-1)
        o_ref[...] = (jnp.dot(pooled, lw_ref[...],
                              preferred_element_type=jnp.float32)
                      + lb_ref[...])


def _pool_readout(S, hp, dis, b, a, beta, batch, linW, linb):
    batch3 = batch.reshape(NBLK, 1, BLK)
    return pl.pallas_call(
        _pool_body,
        grid=(NBLK,),
        in_specs=[pl.BlockSpec((2, BLK, FH), lambda i: (0, i, 0)),
                  pl.BlockSpec((BLK, F), lambda i: (i, 0)),
                  pl.BlockSpec((BLK, 1), lambda i: (i, 0)),
                  pl.BlockSpec((1, F), lambda i: (0, 0)),
                  pl.BlockSpec((1, F), lambda i: (0, 0)),
                  pl.BlockSpec((1, F), lambda i: (0, 0)),
                  pl.BlockSpec((1, 1, BLK), lambda i: (i, 0, 0)),
                  pl.BlockSpec((3 * F, OUT_F), lambda i: (0, 0)),
                  pl.BlockSpec((1, OUT_F), lambda i: (0, 0))],
        out_specs=pl.BlockSpec((G, OUT_F), lambda i: (0, 0)),
        out_shape=jax.ShapeDtypeStruct((G, OUT_F), jnp.float32),
        scratch_shapes=[pltpu.VMEM((G, F), jnp.float32),
                        pltpu.VMEM((G, 1), jnp.float32),
                        pltpu.VMEM((G, F), jnp.float32)],
    )(S, hp, dis, b.reshape(1, F), a.reshape(1, F),
      beta.reshape(1, F), batch3, linW, linb.reshape(1, OUT_F))


_SC_MESH = plsc.VectorSubcoreMesh(core_axis_name="c", subcore_axis_name="s",
                                  num_cores=2, num_subcores=16)
_SC_CP = pltpu.CompilerParams()
if "needs_layout_passes" in pltpu.CompilerParams.__dataclass_fields__:
    _SC_CP = dataclasses.replace(_SC_CP, needs_layout_passes=False)

NCORE = 2
NSUB = 16
NTILES = NCORE * NSUB
EPT = E // NTILES
DCH = 80
NPAD = 10240
RPS = NPAD // NSUB
ZR = 128
SCH = 80
EPS_SUB = E // NSUB
NCHUNK = EPS_SUB // SCH


def _deg_body(col_hbm, ew_hbm, out_hbm, cid_v, ew_v, pay_v, zb_v, acc_sh):
    ci = jax.lax.axis_index("c")
    si = jax.lax.axis_index("s")
    wid = si * NCORE + ci

    zv = jnp.zeros((16,), jnp.float32)
    for i in range(ZR):
        zb_v[i, :] = zv
    for j in range(RPS // ZR):
        pltpu.sync_copy(zb_v, acc_sh.at[pl.ds(si * RPS + j * ZR, ZR)])
    plsc.subcore_barrier()

    base = wid * EPT

    @pl.loop(0, EPT // DCH)
    def _(g):
        off = base + g * DCH
        pltpu.sync_copy(col_hbm.at[pl.ds(off, DCH)], cid_v)
        pltpu.sync_copy(ew_hbm.at[pl.ds(off, DCH)], ew_v)
        for i in range(DCH):
            w = plsc.load_gather(ew_v, [jnp.full((16,), i, jnp.int32)])
            pay_v[i, :] = w
        pltpu.sync_copy(pay_v, acc_sh.at[cid_v], add=True)

    plsc.subcore_barrier()
    pltpu.sync_copy(acc_sh.at[pl.ds(si * RPS, RPS)],
                    out_hbm.at[ci].at[pl.ds(si * RPS, RPS)])


def _deg_scatter(col, ew):
    deg_sc = pl.kernel(
        _deg_body,
        out_type=jax.ShapeDtypeStruct((NCORE, NPAD, 16), jnp.float32),
        mesh=_SC_MESH,
        scratch_types=[
            pltpu.VMEM((DCH,), jnp.int32),
            pltpu.VMEM((DCH,), jnp.float32),
            pltpu.VMEM((DCH, 16), jnp.float32),
            pltpu.VMEM((ZR, 16), jnp.float32),
            pltpu.VMEM_SHARED((NPAD, 16), jnp.float32),
        ],
        compiler_params=_SC_CP,
    )
    partials = deg_sc(col, ew)
    return partials[0, :N, 0] + partials[1, :N, 0] + 1.0


def _spmm_body(hp_hbm, eidx_hbm, z_hbm, out_hbm,
               idxa_v, idxb_v, msga_v, msgb_v, msgh_v, acc_sh,
               ia, ib, ga, gb):
    ci = jax.lax.axis_index("c")
    si = jax.lax.axis_index("s")
    coff = ci * FH

    pltpu.sync_copy(z_hbm, acc_sh.at[pl.ds(si * RPS, RPS)])
    plsc.subcore_barrier()

    def scale(idx_v, msg_v):
        for i in range(SCH):
            w = plsc.bitcast(
                plsc.load_gather(idx_v, [jnp.full((16,), 2, jnp.int32),
                                         jnp.full((16,), i, jnp.int32)]),
                jnp.float32)
            for b in range(FH // 16):
                msgh_v[i, pl.ds(b * 16, 16)] = (
                    msg_v[i, pl.ds(coff + b * 16, 16)] * w)

    H = NCHUNK // 2
    pltpu.sync_copy(eidx_hbm.at[si].at[0], idxa_v)
    pltpu.async_copy(hp_hbm.at[idxa_v.at[0]], msga_v, ga)
    pltpu.async_copy(eidx_hbm.at[si].at[1], idxb_v, ib)

    @pl.loop(0, H)
    def _(h):
        pltpu.make_async_copy(hp_hbm.at[idxa_v.at[0]], msga_v, ga).wait()
        pltpu.make_async_copy(eidx_hbm.at[si].at[0], idxb_v, ib).wait()
        pltpu.async_copy(hp_hbm.at[idxb_v.at[0]], msgb_v, gb)
        scale(idxa_v, msga_v)
        pltpu.sync_copy(msgh_v, acc_sh.at[idxa_v.at[1]], add=True)

        @pl.when(h < H - 1)
        def _():
            pltpu.async_copy(eidx_hbm.at[si].at[2 * h + 2], idxa_v, ia)

        pltpu.make_async_copy(hp_hbm.at[idxb_v.at[0]], msgb_v, gb).wait()

        @pl.when(h < H - 1)
        def _():
            pltpu.make_async_copy(eidx_hbm.at[si].at[0], idxa_v, ia).wait()
            pltpu.async_copy(hp_hbm.at[idxa_v.at[0]], msga_v, ga)

        scale(idxb_v, msgb_v)
        pltpu.sync_copy(msgh_v, acc_sh.at[idxb_v.at[1]], add=True)

        @pl.when(h < H - 1)
        def _():
            pltpu.async_copy(eidx_hbm.at[si].at[2 * h + 3], idxb_v, ib)

    plsc.subcore_barrier()
    pltpu.sync_copy(acc_sh.at[pl.ds(si * RPS, RPS)],
                    out_hbm.at[ci].at[pl.ds(si * RPS, RPS)])


def _spmm_sc(hp, eidx, zrows):
    spmm = pl.kernel(
        _spmm_body,
        out_type=jax.ShapeDtypeStruct((NCORE, NPAD, FH), jnp.float32),
        mesh=_SC_MESH,
        scratch_types=[
            pltpu.VMEM((3, SCH), jnp.int32),
            pltpu.VMEM((3, SCH), jnp.int32),
            pltpu.VMEM((SCH, F), jnp.float32),
            pltpu.VMEM((SCH, F), jnp.float32),
            pltpu.VMEM((SCH, FH), jnp.float32),
            pltpu.VMEM_SHARED((NPAD, FH), jnp.float32),
            pltpu.SemaphoreType.DMA,
            pltpu.SemaphoreType.DMA,
            pltpu.SemaphoreType.DMA,
            pltpu.SemaphoreType.DMA,
        ],
        compiler_params=_SC_CP,
    )
    return spmm(hp, eidx, zrows)


def kernel(x, edge_index, edge_weight, batch, W1, b1, gamma1, beta1,
           W2, b2, gamma2, beta2, linW, linb):
    row = edge_index[0]
    col = edge_index[1]
    bnscale = jax.lax.rsqrt(jnp.float32(1.0 + EPS))
    a1 = gamma1 * bnscale
    a2 = gamma2 * bnscale

    deg = _deg_scatter(col, edge_weight)
    dis = jax.lax.rsqrt(deg).reshape(N, 1)

    eidx = jnp.stack([row.reshape(NSUB, NCHUNK, SCH),
                      col.reshape(NSUB, NCHUNK, SCH),
                      jax.lax.bitcast_convert_type(
                          edge_weight, jnp.int32).reshape(NSUB, NCHUNK, SCH)],
                     axis=2)
    zrows = jnp.zeros((RPS, FH), jnp.float32)

    hw1 = _mm(x, W1)
    h1p = _scale(hw1, dis)

    pstack = (jnp.stack([b1, b2]), jnp.stack([a1, a2]),
              jnp.stack([beta1, beta2]))

    def layer_body(hp, params):
        b_, a_, be_ = params
        S = _spmm_sc(hp, eidx, zrows)
        hp_next = _combine_mm(S, hp, dis, b_, a_, be_, W2)
        return hp_next, (S, hp)

    _, (Ss, hps) = jax.lax.scan(layer_body, h1p, pstack)
    S2 = Ss[1]
    h2p = hps[1]
    return _pool_readout(S2, h2p, dis, b2, a2, beta2, batch, linW, linb)
